# baseline (speedup 1.0000x reference)
import jax
import jax.numpy as jnp
from jax import lax
from jax.experimental import pallas as pl
from jax.experimental.pallas import tpu as pltpu
from jax.experimental.pallas import tpu_sc as plsc

_B = 4096
_CTX = 20
_S = 21
_EMB = 64
_VOCAB = 1000000
_NC = 2
_NS = 16
_NW = _NC * _NS
_BPW = _B // _NW


def _body(d_hbm, w_hbm, wp_hbm, docs_hbm, ctxs_hbm, y_hbm, out_hbm,
          idx_v, ident_v, ctxi_v, yi_v, rows_v, h_v, ht_v, gidx_v, g_v,
          out_v, acc_spm, sem, gsem):
  sid = lax.axis_index("s")
  wid = sid * _NC + lax.axis_index("c")
  base = wid * _BPW
  iota = lax.iota(jnp.int32, 16)
  sbase = sid * _BPW

  for j in range(_BPW // 16):
    ident_v[pl.ds(j * 16, 16)] = iota + (sbase + j * 16)

  pltpu.sync_copy(docs_hbm.at[pl.ds(base, _BPW)], idx_v)
  pltpu.sync_copy(ctxs_hbm.at[pl.ds(base, _BPW), :], ctxi_v)
  pltpu.sync_copy(y_hbm.at[pl.ds(base, _BPW), :], yi_v)

  pltpu.async_copy(d_hbm.at[idx_v], rows_v, sem).wait()
  pltpu.sync_copy(rows_v, acc_spm.at[pl.ds(sbase, _BPW)])

  for c in range(_CTX):
    cc = jnp.zeros((16,), jnp.int32) + c
    for j in range(_BPW // 16):
      idx_v[pl.ds(j * 16, 16)] = plsc.load_gather(
          ctxi_v, [j * 16 + iota, cc])
    pltpu.async_copy(w_hbm.at[idx_v], rows_v, sem).wait()
    pltpu.sync_copy(rows_v, acc_spm.at[ident_v], add=True)

  pltpu.sync_copy(acc_spm.at[pl.ds(sbase, _BPW)], h_v)

  def ht_body(e, _):
    ee = jnp.zeros((16,), jnp.int32) + e
    for j in range(_BPW // 16):
      ht_v[e, pl.ds(j * 16, 16)] = plsc.load_gather(
          h_v, [j * 16 + iota, ee])
    return 0
  lax.fori_loop(0, _EMB, ht_body, 0)

  for s in range(_S):
    ss = jnp.zeros((16,), jnp.int32) + s
    yv = [plsc.load_gather(yi_v, [j * 16 + iota, ss])
          for j in range(_BPW // 16)]

    def gidx_body(e, _):
      off = e * _VOCAB
      for j in range(_BPW // 16):
        gidx_v[e, pl.ds(j * 16, 16)] = yv[j] + off
      return 0
    lax.fori_loop(0, _EMB, gidx_body, 0)

    def fire_body(e, _):
      pltpu.async_copy(wp_hbm.at[gidx_v.at[e]], g_v.at[e], gsem)
      return 0
    lax.fori_loop(0, _EMB, fire_body, 0)

    def drain_body(e, _):
      pltpu.make_async_copy(wp_hbm.at[gidx_v.at[e]], g_v.at[e], gsem).wait()
      return 0
    lax.fori_loop(0, _EMB, drain_body, 0)

    def blk_body(blk, _):
      def e_body(e, acc):
        sl = pl.ds(blk * 16, 16)
        return acc + ht_v[e, sl] * g_v[e, sl]
      acc = lax.fori_loop(0, _EMB, e_body, jnp.zeros((16,), jnp.float32))
      plsc.store_scatter(out_v, [blk * 16 + iota, ss], acc)
      return 0
    lax.fori_loop(0, _BPW // 16, blk_body, 0)

  pltpu.sync_copy(out_v, out_hbm.at[pl.ds(base, _BPW), :])


@jax.jit
def _dmm_call(d, w, wp_flat, docs, ctxs, y):
  mesh = plsc.VectorSubcoreMesh(
      core_axis_name="c", subcore_axis_name="s",
      num_cores=_NC, num_subcores=_NS)
  return pl.kernel(
      _body,
      out_type=jax.ShapeDtypeStruct((_B, _S), jnp.float32),
      mesh=mesh,
      compiler_params=pltpu.CompilerParams(needs_layout_passes=False,
                                           use_tc_tiling_on_sc=False),
      scratch_types=[
          pltpu.VMEM((_BPW,), jnp.int32),
          pltpu.VMEM((_BPW,), jnp.int32),
          pltpu.VMEM((_BPW, _CTX), jnp.int32),
          pltpu.VMEM((_BPW, _S), jnp.int32),
          pltpu.VMEM((_BPW, _EMB), jnp.float32),
          pltpu.VMEM((_BPW, _EMB), jnp.float32),
          pltpu.VMEM((_EMB, _BPW), jnp.float32),
          pltpu.VMEM((_EMB, _BPW), jnp.int32),
          pltpu.VMEM((_EMB, _BPW), jnp.float32),
          pltpu.VMEM((_BPW, _S), jnp.float32),
          pltpu.VMEM_SHARED((_NS * _BPW, _EMB), jnp.float32),
          pltpu.SemaphoreType.DMA,
          pltpu.SemaphoreType.DMA,
      ],
  )(d, w, wp_flat, docs, ctxs, y)


def kernel(D, W, WP, ctxs, docs, y):
  return _dmm_call(D, W, WP.reshape(-1), docs.astype(jnp.int32),
                   ctxs.astype(jnp.int32), y.astype(jnp.int32))


# speedup vs baseline: 3.7857x; 3.7857x over previous
import jax
import jax.numpy as jnp
from jax import lax
from jax.experimental import pallas as pl
from jax.experimental.pallas import tpu as pltpu
from jax.experimental.pallas import tpu_sc as plsc

_B = 4096
_CTX = 20
_S = 21
_EMB = 64
_VOCAB = 1000000
_NC = 2
_NS = 16
_NW = _NC * _NS
_BPW = _B // _NW
_CB = 8192

_SC_PARAMS = pltpu.CompilerParams(needs_layout_passes=False,
                                  use_tc_tiling_on_sc=False)


def _tr_body(wp_ref, out_ref):
  eye = (lax.broadcasted_iota(jnp.int32, (_EMB, _EMB), 0) ==
         lax.broadcasted_iota(jnp.int32, (_EMB, _EMB), 1)).astype(jnp.float32)
  out_ref[...] = lax.dot_general(
      wp_ref[...], eye, (((0,), (0,)), ((), ())),
      preferred_element_type=jnp.float32,
      precision=lax.Precision.HIGHEST)


@jax.jit
def _transpose_call(wp):
  nb = (_VOCAB + _CB - 1) // _CB
  return pl.pallas_call(
      _tr_body,
      out_shape=jax.ShapeDtypeStruct((_VOCAB, _EMB), jnp.float32),
      grid=(nb,),
      in_specs=[pl.BlockSpec((_EMB, _CB), lambda i: (0, i))],
      out_specs=pl.BlockSpec((_CB, _EMB), lambda i: (i, 0)),
  )(wp)


def _sc1_body(d_hbm, w_hbm, docs_hbm, ctxs_hbm, h_hbm,
              idx_a, idx_b, ident_v, ctxi_v, rows_a, rows_b, acc_spm,
              sem_a, sem_b):
  sid = lax.axis_index("s")
  wid = sid * _NC + lax.axis_index("c")
  base = wid * _BPW
  iota = lax.iota(jnp.int32, 16)
  sbase = sid * _BPW

  for j in range(_BPW // 16):
    ident_v[pl.ds(j * 16, 16)] = iota + (sbase + j * 16)

  idx = (idx_a, idx_b)
  rows = (rows_a, rows_b)
  sems = (sem_a, sem_b)

  pltpu.sync_copy(docs_hbm.at[pl.ds(base, _BPW)], idx_a)
  pltpu.async_copy(d_hbm.at[idx_a], rows_a, sem_a)
  pltpu.sync_copy(ctxs_hbm.at[pl.ds(base, _BPW), :], ctxi_v)

  def build_idx(c, dst):
    cc = jnp.zeros((16,), jnp.int32) + c
    for j in range(_BPW // 16):
      dst[pl.ds(j * 16, 16)] = plsc.load_gather(ctxi_v, [j * 16 + iota, cc])

  build_idx(0, idx_b)
  pltpu.make_async_copy(d_hbm.at[idx_a], rows_a, sem_a).wait()
  pltpu.sync_copy(rows_a, acc_spm.at[pl.ds(sbase, _BPW)])
  pltpu.async_copy(w_hbm.at[idx_b], rows_b, sem_b)

  for c in range(_CTX):
    cur = (c + 1) % 2
    nxt = c % 2
    if c + 1 < _CTX:
      build_idx(c + 1, idx[nxt])
      pltpu.async_copy(w_hbm.at[idx[nxt]], rows[nxt], sems[nxt])
    pltpu.make_async_copy(w_hbm.at[idx[cur]], rows[cur], sems[cur]).wait()
    pltpu.sync_copy(rows[cur], acc_spm.at[ident_v], add=True)

  pltpu.sync_copy(acc_spm.at[pl.ds(sbase, _BPW)],
                  h_hbm.at[pl.ds(base, _BPW), :])


@jax.jit
def _sc1_call(d, w, docs, ctxs):
  mesh = plsc.VectorSubcoreMesh(
      core_axis_name="c", subcore_axis_name="s",
      num_cores=_NC, num_subcores=_NS)
  return pl.kernel(
      _sc1_body,
      out_type=jax.ShapeDtypeStruct((_B, _EMB), jnp.float32),
      mesh=mesh,
      compiler_params=_SC_PARAMS,
      scratch_types=[
          pltpu.VMEM((_BPW,), jnp.int32),
          pltpu.VMEM((_BPW,), jnp.int32),
          pltpu.VMEM((_BPW,), jnp.int32),
          pltpu.VMEM((_BPW, _CTX), jnp.int32),
          pltpu.VMEM((_BPW, _EMB), jnp.float32),
          pltpu.VMEM((_BPW, _EMB), jnp.float32),
          pltpu.VMEM_SHARED((_NS * _BPW, _EMB), jnp.float32),
          pltpu.SemaphoreType.DMA,
          pltpu.SemaphoreType.DMA,
      ],
  )(d, w, docs, ctxs)


def _sc2_body(h_hbm, wpt_hbm, y_hbm, out_hbm,
              yi_v, yidx_a, yidx_b, h_v, g_a, g_b, out_v, sem_a, sem_b):
  sid = lax.axis_index("s")
  wid = sid * _NC + lax.axis_index("c")
  base = wid * _BPW
  iota = lax.iota(jnp.int32, 16)
  lane15 = iota == 15

  yidx = (yidx_a, yidx_b)
  g = (g_a, g_b)
  sems = (sem_a, sem_b)

  pltpu.sync_copy(y_hbm.at[pl.ds(base, _BPW), :], yi_v)
  pltpu.sync_copy(h_hbm.at[pl.ds(base, _BPW), :], h_v)

  def build_idx(s, dst):
    ss = jnp.zeros((16,), jnp.int32) + s
    for j in range(_BPW // 16):
      dst[pl.ds(j * 16, 16)] = plsc.load_gather(yi_v, [j * 16 + iota, ss])

  build_idx(0, yidx_a)
  pltpu.async_copy(wpt_hbm.at[yidx_a], g_a, sem_a)

  for s in range(_S):
    cur = s % 2
    nxt = (s + 1) % 2
    if s + 1 < _S:
      build_idx(s + 1, yidx[nxt])
      pltpu.async_copy(wpt_hbm.at[yidx[nxt]], g[nxt], sems[nxt])
    pltpu.make_async_copy(wpt_hbm.at[yidx[cur]], g[cur], sems[cur]).wait()

    gc = g[cur]
    ss = jnp.zeros((16,), jnp.int32) + s

    def dot_row(i, _):
      acc = h_v[i, pl.ds(0, 16)] * gc[i, pl.ds(0, 16)]
      for j in range(1, _EMB // 16):
        sl = pl.ds(j * 16, 16)
        acc = acc + h_v[i, sl] * gc[i, sl]
      csum = plsc.cumsum(acc)
      plsc.store_scatter(out_v, [jnp.zeros((16,), jnp.int32) + i, ss],
                         csum, mask=lane15)
      return 0
    lax.fori_loop(0, _BPW, dot_row, 0)

  pltpu.sync_copy(out_v, out_hbm.at[pl.ds(base, _BPW), :])


@jax.jit
def _sc2_call(h, wpt, y):
  mesh = plsc.VectorSubcoreMesh(
      core_axis_name="c", subcore_axis_name="s",
      num_cores=_NC, num_subcores=_NS)
  return pl.kernel(
      _sc2_body,
      out_type=jax.ShapeDtypeStruct((_B, _S), jnp.float32),
      mesh=mesh,
      compiler_params=_SC_PARAMS,
      scratch_types=[
          pltpu.VMEM((_BPW, _S), jnp.int32),
          pltpu.VMEM((_BPW,), jnp.int32),
          pltpu.VMEM((_BPW,), jnp.int32),
          pltpu.VMEM((_BPW, _EMB), jnp.float32),
          pltpu.VMEM((_BPW, _EMB), jnp.float32),
          pltpu.VMEM((_BPW, _EMB), jnp.float32),
          pltpu.VMEM((_BPW, _S), jnp.float32),
          pltpu.SemaphoreType.DMA,
          pltpu.SemaphoreType.DMA,
      ],
  )(h, wpt, y)


def kernel(D, W, WP, ctxs, docs, y):
  wpt = _transpose_call(WP)
  h = _sc1_call(D, W, docs.astype(jnp.int32), ctxs.astype(jnp.int32))
  return _sc2_call(h, wpt, y.astype(jnp.int32))


# speedup vs baseline: 5.0723x; 1.3398x over previous
import jax
import jax.numpy as jnp
from jax import lax
from jax.experimental import pallas as pl
from jax.experimental.pallas import tpu as pltpu
from jax.experimental.pallas import tpu_sc as plsc

_B = 4096
_CTX = 20
_S = 21
_EMB = 64
_LANES = 128
_VOCAB = 1000000
_NC = 2
_NS = 16
_NW = _NC * _NS
_BPW = _B // _NW
_CB = 8192

_SC_PARAMS = pltpu.CompilerParams(needs_layout_passes=False,
                                  use_tc_tiling_on_sc=False)


def _tr_body(wp_ref, out_ref):
  eye = (lax.broadcasted_iota(jnp.int32, (_EMB, _LANES), 0) ==
         lax.broadcasted_iota(jnp.int32, (_EMB, _LANES), 1)).astype(jnp.float32)
  out_ref[...] = lax.dot_general(
      wp_ref[...], eye, (((0,), (0,)), ((), ())),
      preferred_element_type=jnp.float32,
      precision=lax.Precision.HIGHEST)


@jax.jit
def _transpose_call(wp):
  nb = (_VOCAB + _CB - 1) // _CB
  return pl.pallas_call(
      _tr_body,
      out_shape=jax.ShapeDtypeStruct((_VOCAB, _LANES), jnp.float32),
      grid=(nb,),
      in_specs=[pl.BlockSpec((_EMB, _CB), lambda i: (0, i))],
      out_specs=pl.BlockSpec((_CB, _LANES), lambda i: (i, 0)),
  )(wp)


def _sc1_body(d_hbm, w_hbm, docs_hbm, ctxs_hbm, h_hbm,
              idx_a, idx_b, ident_v, ctxi_v, rows_a, rows_b, acc_spm,
              sem_a, sem_b):
  sid = lax.axis_index("s")
  wid = sid * _NC + lax.axis_index("c")
  base = wid * _BPW
  iota = lax.iota(jnp.int32, 16)
  sbase = sid * _BPW

  for j in range(_BPW // 16):
    ident_v[pl.ds(j * 16, 16)] = iota + (sbase + j * 16)

  idx = (idx_a, idx_b)
  rows = (rows_a, rows_b)
  sems = (sem_a, sem_b)

  pltpu.sync_copy(docs_hbm.at[pl.ds(base, _BPW)], idx_a)
  pltpu.async_copy(d_hbm.at[idx_a], rows_a, sem_a)
  pltpu.sync_copy(ctxs_hbm.at[pl.ds(base, _BPW), :], ctxi_v)

  def build_idx(c, dst):
    cc = jnp.zeros((16,), jnp.int32) + c
    for j in range(_BPW // 16):
      dst[pl.ds(j * 16, 16)] = plsc.load_gather(ctxi_v, [j * 16 + iota, cc])

  build_idx(0, idx_b)
  pltpu.make_async_copy(d_hbm.at[idx_a], rows_a, sem_a).wait()
  pltpu.sync_copy(rows_a, acc_spm.at[pl.ds(sbase, _BPW)])
  pltpu.async_copy(w_hbm.at[idx_b], rows_b, sem_b)

  for c in range(_CTX):
    cur = (c + 1) % 2
    nxt = c % 2
    if c + 1 < _CTX:
      build_idx(c + 1, idx[nxt])
      pltpu.async_copy(w_hbm.at[idx[nxt]], rows[nxt], sems[nxt])
    pltpu.make_async_copy(w_hbm.at[idx[cur]], rows[cur], sems[cur]).wait()
    pltpu.sync_copy(rows[cur], acc_spm.at[ident_v], add=True)

  pltpu.sync_copy(acc_spm.at[pl.ds(sbase, _BPW)],
                  h_hbm.at[pl.ds(base, _BPW), :])


@jax.jit
def _sc1_call(d, w, docs, ctxs):
  mesh = plsc.VectorSubcoreMesh(
      core_axis_name="c", subcore_axis_name="s",
      num_cores=_NC, num_subcores=_NS)
  return pl.kernel(
      _sc1_body,
      out_type=jax.ShapeDtypeStruct((_B, _EMB), jnp.float32),
      mesh=mesh,
      compiler_params=_SC_PARAMS,
      scratch_types=[
          pltpu.VMEM((_BPW,), jnp.int32),
          pltpu.VMEM((_BPW,), jnp.int32),
          pltpu.VMEM((_BPW,), jnp.int32),
          pltpu.VMEM((_BPW, _CTX), jnp.int32),
          pltpu.VMEM((_BPW, _EMB), jnp.float32),
          pltpu.VMEM((_BPW, _EMB), jnp.float32),
          pltpu.VMEM_SHARED((_NS * _BPW, _EMB), jnp.float32),
          pltpu.SemaphoreType.DMA,
          pltpu.SemaphoreType.DMA,
      ],
  )(d, w, docs, ctxs)


def _sc2_body(h_hbm, wpt_hbm, y_hbm, out_hbm,
              yi_v, yidx_a, yidx_b, h_v, g_a, g_b, out_v, sem_a, sem_b):
  sid = lax.axis_index("s")
  wid = sid * _NC + lax.axis_index("c")
  base = wid * _BPW
  iota = lax.iota(jnp.int32, 16)
  lane15 = iota == 15

  yidx = (yidx_a, yidx_b)
  g = (g_a, g_b)
  sems = (sem_a, sem_b)

  pltpu.sync_copy(y_hbm.at[pl.ds(base, _BPW), :], yi_v)
  pltpu.sync_copy(h_hbm.at[pl.ds(base, _BPW), :], h_v)

  def build_idx(s, dst):
    ss = jnp.zeros((16,), jnp.int32) + s
    for j in range(_BPW // 16):
      dst[pl.ds(j * 16, 16)] = plsc.load_gather(yi_v, [j * 16 + iota, ss])

  build_idx(0, yidx_a)
  pltpu.async_copy(wpt_hbm.at[yidx_a], g_a, sem_a)

  for s in range(_S):
    cur = s % 2
    nxt = (s + 1) % 2
    if s + 1 < _S:
      build_idx(s + 1, yidx[nxt])
      pltpu.async_copy(wpt_hbm.at[yidx[nxt]], g[nxt], sems[nxt])
    pltpu.make_async_copy(wpt_hbm.at[yidx[cur]], g[cur], sems[cur]).wait()

    gc = g[cur]
    ss = jnp.zeros((16,), jnp.int32) + s

    def dot_row(i, _):
      acc = h_v[i, pl.ds(0, 16)] * gc[i, pl.ds(0, 16)]
      for j in range(1, _EMB // 16):
        sl = pl.ds(j * 16, 16)
        acc = acc + h_v[i, sl] * gc[i, sl]
      csum = plsc.cumsum(acc)
      plsc.store_scatter(out_v, [jnp.zeros((16,), jnp.int32) + i, ss],
                         csum, mask=lane15)
      return 0
    lax.fori_loop(0, _BPW, dot_row, 0)

  pltpu.sync_copy(out_v, out_hbm.at[pl.ds(base, _BPW), :])


@jax.jit
def _sc2_call(h, wpt, y):
  mesh = plsc.VectorSubcoreMesh(
      core_axis_name="c", subcore_axis_name="s",
      num_cores=_NC, num_subcores=_NS)
  return pl.kernel(
      _sc2_body,
      out_type=jax.ShapeDtypeStruct((_B, _S), jnp.float32),
      mesh=mesh,
      compiler_params=_SC_PARAMS,
      scratch_types=[
          pltpu.VMEM((_BPW, _S), jnp.int32),
          pltpu.VMEM((_BPW,), jnp.int32),
          pltpu.VMEM((_BPW,), jnp.int32),
          pltpu.VMEM((_BPW, _EMB), jnp.float32),
          pltpu.VMEM((_BPW, _LANES), jnp.float32),
          pltpu.VMEM((_BPW, _LANES), jnp.float32),
          pltpu.VMEM((_BPW, _S), jnp.float32),
          pltpu.SemaphoreType.DMA,
          pltpu.SemaphoreType.DMA,
      ],
  )(h, wpt, y)


def kernel(D, W, WP, ctxs, docs, y):
  wpt = _transpose_call(WP)
  h = _sc1_call(D, W, docs.astype(jnp.int32), ctxs.astype(jnp.int32))
  return _sc2_call(h, wpt, y.astype(jnp.int32))


# speedup vs baseline: 5.8135x; 1.1461x over previous
import jax
import jax.numpy as jnp
from jax import lax
from jax.experimental import pallas as pl
from jax.experimental.pallas import tpu as pltpu
from jax.experimental.pallas import tpu_sc as plsc

_B = 4096
_CTX = 20
_S = 21
_EMB = 64
_LANES = 128
_VOCAB = 1000000
_NC = 2
_NS = 16
_NW = _NC * _NS
_BPW = _B // _NW
_CB = 8192

_SC_PARAMS = pltpu.CompilerParams(needs_layout_passes=False,
                                  use_tc_tiling_on_sc=False)


def _tr_body(x_ref, out_ref):
  eye = (lax.broadcasted_iota(jnp.int32, (_EMB, _LANES), 0) ==
         lax.broadcasted_iota(jnp.int32, (_EMB, _LANES), 1)).astype(jnp.float32)
  out_ref[...] = lax.dot_general(
      x_ref[...], eye, (((0,), (0,)), ((), ())),
      preferred_element_type=jnp.float32,
      precision=lax.Precision.HIGHEST)


def _to_rows(xt):
  n = xt.shape[1]
  nb = (n + _CB - 1) // _CB
  return pl.pallas_call(
      _tr_body,
      out_shape=jax.ShapeDtypeStruct((nb * _CB, _LANES), jnp.float32),
      grid=(nb,),
      in_specs=[pl.BlockSpec((_EMB, _CB), lambda i: (0, i))],
      out_specs=pl.BlockSpec((_CB, _LANES), lambda i: (i, 0)),
  )(xt)


def _sc1_body(d_hbm, w_hbm, docs_hbm, ctxs_hbm, h_hbm,
              idx_a, idx_b, ident_v, ctxi_v, rows_a, rows_b, acc_spm,
              sem_a, sem_b):
  sid = lax.axis_index("s")
  wid = sid * _NC + lax.axis_index("c")
  base = wid * _BPW
  iota = lax.iota(jnp.int32, 16)
  sbase = sid * _BPW

  for j in range(_BPW // 16):
    ident_v[pl.ds(j * 16, 16)] = iota + (sbase + j * 16)

  idx = (idx_a, idx_b)
  rows = (rows_a, rows_b)
  sems = (sem_a, sem_b)

  pltpu.sync_copy(docs_hbm.at[pl.ds(base, _BPW)], idx_a)
  pltpu.async_copy(d_hbm.at[idx_a], rows_a, sem_a)
  pltpu.sync_copy(ctxs_hbm.at[pl.ds(base, _BPW), :], ctxi_v)

  def build_idx(c, dst):
    cc = jnp.zeros((16,), jnp.int32) + c
    for j in range(_BPW // 16):
      dst[pl.ds(j * 16, 16)] = plsc.load_gather(ctxi_v, [j * 16 + iota, cc])

  build_idx(0, idx_b)
  pltpu.make_async_copy(d_hbm.at[idx_a], rows_a, sem_a).wait()
  pltpu.sync_copy(rows_a, acc_spm.at[pl.ds(sbase, _BPW)])
  pltpu.async_copy(w_hbm.at[idx_b], rows_b, sem_b)

  for c in range(_CTX):
    cur = (c + 1) % 2
    nxt = c % 2
    if c + 1 < _CTX:
      build_idx(c + 1, idx[nxt])
      pltpu.async_copy(w_hbm.at[idx[nxt]], rows[nxt], sems[nxt])
    pltpu.make_async_copy(w_hbm.at[idx[cur]], rows[cur], sems[cur]).wait()
    pltpu.sync_copy(rows[cur], acc_spm.at[ident_v], add=True)

  pltpu.sync_copy(acc_spm.at[pl.ds(sbase, _BPW)],
                  h_hbm.at[pl.ds(base, _BPW), :])


@jax.jit
def _sc1_call(d, w, docs, ctxs):
  mesh = plsc.VectorSubcoreMesh(
      core_axis_name="c", subcore_axis_name="s",
      num_cores=_NC, num_subcores=_NS)
  return pl.kernel(
      _sc1_body,
      out_type=jax.ShapeDtypeStruct((_B, _LANES), jnp.float32),
      mesh=mesh,
      compiler_params=_SC_PARAMS,
      scratch_types=[
          pltpu.VMEM((_BPW,), jnp.int32),
          pltpu.VMEM((_BPW,), jnp.int32),
          pltpu.VMEM((_BPW,), jnp.int32),
          pltpu.VMEM((_BPW, _CTX), jnp.int32),
          pltpu.VMEM((_BPW, _LANES), jnp.float32),
          pltpu.VMEM((_BPW, _LANES), jnp.float32),
          pltpu.VMEM_SHARED((_NS * _BPW, _LANES), jnp.float32),
          pltpu.SemaphoreType.DMA,
          pltpu.SemaphoreType.DMA,
      ],
  )(d, w, docs, ctxs)


def _sc2_body(h_hbm, wpt_hbm, y_hbm, out_hbm,
              yi_v, yidx_a, yidx_b, h_v, g_a, g_b, out_v, sem_a, sem_b):
  sid = lax.axis_index("s")
  wid = sid * _NC + lax.axis_index("c")
  base = wid * _BPW
  iota = lax.iota(jnp.int32, 16)
  lane15 = iota == 15

  yidx = (yidx_a, yidx_b)
  g = (g_a, g_b)
  sems = (sem_a, sem_b)

  pltpu.sync_copy(y_hbm.at[pl.ds(base, _BPW), :], yi_v)
  pltpu.sync_copy(h_hbm.at[pl.ds(base, _BPW), :], h_v)

  def build_idx(s, dst):
    ss = jnp.zeros((16,), jnp.int32) + s
    for j in range(_BPW // 16):
      dst[pl.ds(j * 16, 16)] = plsc.load_gather(yi_v, [j * 16 + iota, ss])

  build_idx(0, yidx_a)
  pltpu.async_copy(wpt_hbm.at[yidx_a], g_a, sem_a)

  for s in range(_S):
    cur = s % 2
    nxt = (s + 1) % 2
    if s + 1 < _S:
      build_idx(s + 1, yidx[nxt])
      pltpu.async_copy(wpt_hbm.at[yidx[nxt]], g[nxt], sems[nxt])
    pltpu.make_async_copy(wpt_hbm.at[yidx[cur]], g[cur], sems[cur]).wait()

    gc = g[cur]
    ss = jnp.zeros((16,), jnp.int32) + s

    def dot_row(i, _):
      acc = h_v[i, pl.ds(0, 16)] * gc[i, pl.ds(0, 16)]
      for j in range(1, _EMB // 16):
        sl = pl.ds(j * 16, 16)
        acc = acc + h_v[i, sl] * gc[i, sl]
      csum = plsc.cumsum(acc)
      plsc.store_scatter(out_v, [jnp.zeros((16,), jnp.int32) + i, ss],
                         csum, mask=lane15)
      return 0
    lax.fori_loop(0, _BPW, dot_row, 0)

  pltpu.sync_copy(out_v, out_hbm.at[pl.ds(base, _BPW), :])


@jax.jit
def _sc2_call(h, wpt, y):
  mesh = plsc.VectorSubcoreMesh(
      core_axis_name="c", subcore_axis_name="s",
      num_cores=_NC, num_subcores=_NS)
  return pl.kernel(
      _sc2_body,
      out_type=jax.ShapeDtypeStruct((_B, _S), jnp.float32),
      mesh=mesh,
      compiler_params=_SC_PARAMS,
      scratch_types=[
          pltpu.VMEM((_BPW, _S), jnp.int32),
          pltpu.VMEM((_BPW,), jnp.int32),
          pltpu.VMEM((_BPW,), jnp.int32),
          pltpu.VMEM((_BPW, _LANES), jnp.float32),
          pltpu.VMEM((_BPW, _LANES), jnp.float32),
          pltpu.VMEM((_BPW, _LANES), jnp.float32),
          pltpu.VMEM((_BPW, _S), jnp.float32),
          pltpu.SemaphoreType.DMA,
          pltpu.SemaphoreType.DMA,
      ],
  )(h, wpt, y)


def kernel(D, W, WP, ctxs, docs, y):
  wpt = _to_rows(WP)
  w_lin = _to_rows(W.T)
  d_lin = _to_rows(D.T)
  h = _sc1_call(d_lin, w_lin, docs.astype(jnp.int32), ctxs.astype(jnp.int32))
  return _sc2_call(h, wpt, y.astype(jnp.int32))


# speedup vs baseline: 8.9528x; 1.5400x over previous
import jax
import jax.numpy as jnp
from jax import lax
from jax.experimental import pallas as pl
from jax.experimental.pallas import tpu as pltpu
from jax.experimental.pallas import tpu_sc as plsc

_B = 4096
_CTX = 20
_S = 21
_EMB = 64
_LANES = 128
_VOCAB = 1000000
_NC = 2
_NS = 16
_NW = _NC * _NS
_BPW = _B // _NW
_CB = 8192

_SC_PARAMS = pltpu.CompilerParams(needs_layout_passes=False,
                                  use_tc_tiling_on_sc=False)


def _tr_body(x_ref, out_ref):
  out_ref[:, 0:_EMB] = x_ref[...].T
  out_ref[:, _EMB:_LANES] = jnp.zeros((_CB, _LANES - _EMB), jnp.float32)


def _to_rows(xt):
  n = xt.shape[1]
  nb = (n + _CB - 1) // _CB
  return pl.pallas_call(
      _tr_body,
      out_shape=jax.ShapeDtypeStruct((nb * _CB, _LANES), jnp.float32),
      grid=(nb,),
      in_specs=[pl.BlockSpec((_EMB, _CB), lambda i: (0, i))],
      out_specs=pl.BlockSpec((_CB, _LANES), lambda i: (i, 0)),
  )(xt)


def _sc1_body(d_hbm, w_hbm, docs_hbm, ctxs_hbm, h_hbm,
              idx_a, idx_b, ident_v, ctxi_v, rows_a, rows_b, acc_spm,
              sem_a, sem_b):
  sid = lax.axis_index("s")
  wid = sid * _NC + lax.axis_index("c")
  base = wid * _BPW
  iota = lax.iota(jnp.int32, 16)
  sbase = sid * _BPW

  for j in range(_BPW // 16):
    ident_v[pl.ds(j * 16, 16)] = iota + (sbase + j * 16)

  idx = (idx_a, idx_b)
  rows = (rows_a, rows_b)
  sems = (sem_a, sem_b)

  pltpu.sync_copy(docs_hbm.at[pl.ds(base, _BPW)], idx_a)
  pltpu.async_copy(d_hbm.at[idx_a], rows_a, sem_a)
  pltpu.sync_copy(ctxs_hbm.at[pl.ds(base, _BPW), :], ctxi_v)

  def build_idx(c, dst):
    cc = jnp.zeros((16,), jnp.int32) + c
    for j in range(_BPW // 16):
      dst[pl.ds(j * 16, 16)] = plsc.load_gather(ctxi_v, [j * 16 + iota, cc])

  build_idx(0, idx_b)
  pltpu.make_async_copy(d_hbm.at[idx_a], rows_a, sem_a).wait()
  pltpu.sync_copy(rows_a, acc_spm.at[pl.ds(sbase, _BPW)])
  pltpu.async_copy(w_hbm.at[idx_b], rows_b, sem_b)

  for c in range(_CTX):
    cur = (c + 1) % 2
    nxt = c % 2
    if c + 1 < _CTX:
      build_idx(c + 1, idx[nxt])
      pltpu.async_copy(w_hbm.at[idx[nxt]], rows[nxt], sems[nxt])
    pltpu.make_async_copy(w_hbm.at[idx[cur]], rows[cur], sems[cur]).wait()
    pltpu.sync_copy(rows[cur], acc_spm.at[ident_v], add=True)

  pltpu.sync_copy(acc_spm.at[pl.ds(sbase, _BPW)],
                  h_hbm.at[pl.ds(base, _BPW), :])


@jax.jit
def _sc1_call(d, w, docs, ctxs):
  mesh = plsc.VectorSubcoreMesh(
      core_axis_name="c", subcore_axis_name="s",
      num_cores=_NC, num_subcores=_NS)
  return pl.kernel(
      _sc1_body,
      out_type=jax.ShapeDtypeStruct((_B, _LANES), jnp.float32),
      mesh=mesh,
      compiler_params=_SC_PARAMS,
      scratch_types=[
          pltpu.VMEM((_BPW,), jnp.int32),
          pltpu.VMEM((_BPW,), jnp.int32),
          pltpu.VMEM((_BPW,), jnp.int32),
          pltpu.VMEM((_BPW, _CTX), jnp.int32),
          pltpu.VMEM((_BPW, _LANES), jnp.float32),
          pltpu.VMEM((_BPW, _LANES), jnp.float32),
          pltpu.VMEM_SHARED((_NS * _BPW, _LANES), jnp.float32),
          pltpu.SemaphoreType.DMA,
          pltpu.SemaphoreType.DMA,
      ],
  )(d, w, docs, ctxs)


def _sc2_body(h_hbm, wpt_hbm, y_hbm, out_hbm,
              yi_v, yidx_a, yidx_b, h_v, g_a, g_b, out_v, sem_a, sem_b):
  sid = lax.axis_index("s")
  wid = sid * _NC + lax.axis_index("c")
  base = wid * _BPW
  iota = lax.iota(jnp.int32, 16)
  lane15 = iota == 15

  yidx = (yidx_a, yidx_b)
  g = (g_a, g_b)
  sems = (sem_a, sem_b)

  pltpu.sync_copy(y_hbm.at[pl.ds(base, _BPW), :], yi_v)
  pltpu.sync_copy(h_hbm.at[pl.ds(base, _BPW), :], h_v)

  def build_idx(s, dst):
    ss = jnp.zeros((16,), jnp.int32) + s
    for j in range(_BPW // 16):
      dst[pl.ds(j * 16, 16)] = plsc.load_gather(yi_v, [j * 16 + iota, ss])

  build_idx(0, yidx_a)
  pltpu.async_copy(wpt_hbm.at[yidx_a], g_a, sem_a)

  for s in range(_S):
    cur = s % 2
    nxt = (s + 1) % 2
    if s + 1 < _S:
      build_idx(s + 1, yidx[nxt])
      pltpu.async_copy(wpt_hbm.at[yidx[nxt]], g[nxt], sems[nxt])
    pltpu.make_async_copy(wpt_hbm.at[yidx[cur]], g[cur], sems[cur]).wait()

    gc = g[cur]
    ss = jnp.zeros((16,), jnp.int32) + s

    def dot_row(i, _):
      acc = h_v[i, pl.ds(0, 16)] * gc[i, pl.ds(0, 16)]
      for j in range(1, _EMB // 16):
        sl = pl.ds(j * 16, 16)
        acc = acc + h_v[i, sl] * gc[i, sl]
      csum = plsc.cumsum(acc)
      plsc.store_scatter(out_v, [jnp.zeros((16,), jnp.int32) + i, ss],
                         csum, mask=lane15)
      return 0
    lax.fori_loop(0, _BPW, dot_row, 0)

  pltpu.sync_copy(out_v, out_hbm.at[pl.ds(base, _BPW), :])


@jax.jit
def _sc2_call(h, wpt, y):
  mesh = plsc.VectorSubcoreMesh(
      core_axis_name="c", subcore_axis_name="s",
      num_cores=_NC, num_subcores=_NS)
  return pl.kernel(
      _sc2_body,
      out_type=jax.ShapeDtypeStruct((_B, _S), jnp.float32),
      mesh=mesh,
      compiler_params=_SC_PARAMS,
      scratch_types=[
          pltpu.VMEM((_BPW, _S), jnp.int32),
          pltpu.VMEM((_BPW,), jnp.int32),
          pltpu.VMEM((_BPW,), jnp.int32),
          pltpu.VMEM((_BPW, _LANES), jnp.float32),
          pltpu.VMEM((_BPW, _LANES), jnp.float32),
          pltpu.VMEM((_BPW, _LANES), jnp.float32),
          pltpu.VMEM((_BPW, _S), jnp.float32),
          pltpu.SemaphoreType.DMA,
          pltpu.SemaphoreType.DMA,
      ],
  )(h, wpt, y)


def kernel(D, W, WP, ctxs, docs, y):
  wpt = _to_rows(WP)
  w_lin = _to_rows(W.T)
  d_lin = _to_rows(D.T)
  h = _sc1_call(d_lin, w_lin, docs.astype(jnp.int32), ctxs.astype(jnp.int32))
  return _sc2_call(h, wpt, y.astype(jnp.int32))


# speedup vs baseline: 9.4682x; 1.0576x over previous
import jax
import jax.numpy as jnp
from jax import lax
from jax.experimental import pallas as pl
from jax.experimental.pallas import tpu as pltpu
from jax.experimental.pallas import tpu_sc as plsc

_B = 4096
_CTX = 20
_S = 21
_EMB = 64
_LANES = 128
_VOCAB = 1000000
_NC = 2
_NS = 16
_NW = _NC * _NS
_BPW = _B // _NW
_CB = 16384

_SC_PARAMS = pltpu.CompilerParams(needs_layout_passes=False,
                                  use_tc_tiling_on_sc=False)


def _tr_body(x_ref, out_ref):
  out_ref[:, 0:_EMB] = x_ref[...].T
  out_ref[:, _EMB:_LANES] = jnp.zeros((_CB, _LANES - _EMB), jnp.float32)


def _to_rows(xt):
  n = xt.shape[1]
  nb = (n + _CB - 1) // _CB
  return pl.pallas_call(
      _tr_body,
      out_shape=jax.ShapeDtypeStruct((nb * _CB, _LANES), jnp.float32),
      grid=(nb,),
      in_specs=[pl.BlockSpec((_EMB, _CB), lambda i: (0, i))],
      out_specs=pl.BlockSpec((_CB, _LANES), lambda i: (i, 0)),
  )(xt)


def _sc1_body(d_hbm, w_hbm, docs_hbm, ctxs_hbm, h_hbm,
              idx_a, idx_b, ident_v, ctxi_v, rows_a, rows_b, acc_spm,
              sem_a, sem_b):
  sid = lax.axis_index("s")
  wid = sid * _NC + lax.axis_index("c")
  base = wid * _BPW
  iota = lax.iota(jnp.int32, 16)
  sbase = sid * _BPW

  for j in range(_BPW // 16):
    ident_v[pl.ds(j * 16, 16)] = iota + (sbase + j * 16)

  idx = (idx_a, idx_b)
  rows = (rows_a, rows_b)
  sems = (sem_a, sem_b)

  pltpu.sync_copy(docs_hbm.at[pl.ds(base, _BPW)], idx_a)
  pltpu.async_copy(d_hbm.at[idx_a], rows_a, sem_a)
  pltpu.sync_copy(ctxs_hbm.at[pl.ds(base, _BPW), :], ctxi_v)

  def build_idx(c, dst):
    cc = jnp.zeros((16,), jnp.int32) + c
    for j in range(_BPW // 16):
      dst[pl.ds(j * 16, 16)] = plsc.load_gather(ctxi_v, [j * 16 + iota, cc])

  build_idx(0, idx_b)
  pltpu.make_async_copy(d_hbm.at[idx_a], rows_a, sem_a).wait()
  pltpu.sync_copy(rows_a, acc_spm.at[pl.ds(sbase, _BPW)])
  pltpu.async_copy(w_hbm.at[idx_b], rows_b, sem_b)

  for c in range(_CTX):
    cur = (c + 1) % 2
    nxt = c % 2
    if c + 1 < _CTX:
      build_idx(c + 1, idx[nxt])
      pltpu.async_copy(w_hbm.at[idx[nxt]], rows[nxt], sems[nxt])
    pltpu.make_async_copy(w_hbm.at[idx[cur]], rows[cur], sems[cur]).wait()
    pltpu.sync_copy(rows[cur], acc_spm.at[ident_v], add=True)

  pltpu.sync_copy(acc_spm.at[pl.ds(sbase, _BPW)],
                  h_hbm.at[pl.ds(base, _BPW), :])


@jax.jit
def _sc1_call(d, w, docs, ctxs):
  mesh = plsc.VectorSubcoreMesh(
      core_axis_name="c", subcore_axis_name="s",
      num_cores=_NC, num_subcores=_NS)
  return pl.kernel(
      _sc1_body,
      out_type=jax.ShapeDtypeStruct((_B, _LANES), jnp.float32),
      mesh=mesh,
      compiler_params=_SC_PARAMS,
      scratch_types=[
          pltpu.VMEM((_BPW,), jnp.int32),
          pltpu.VMEM((_BPW,), jnp.int32),
          pltpu.VMEM((_BPW,), jnp.int32),
          pltpu.VMEM((_BPW, _CTX), jnp.int32),
          pltpu.VMEM((_BPW, _LANES), jnp.float32),
          pltpu.VMEM((_BPW, _LANES), jnp.float32),
          pltpu.VMEM_SHARED((_NS * _BPW, _LANES), jnp.float32),
          pltpu.SemaphoreType.DMA,
          pltpu.SemaphoreType.DMA,
      ],
  )(d, w, docs, ctxs)


def _sc2_body(h_hbm, wpt_hbm, y_hbm, out_hbm,
              yi_v, yidx_a, yidx_b, h_v, g_a, g_b, out_v, sem_a, sem_b):
  sid = lax.axis_index("s")
  wid = sid * _NC + lax.axis_index("c")
  base = wid * _BPW
  iota = lax.iota(jnp.int32, 16)
  lane15 = iota == 15

  yidx = (yidx_a, yidx_b)
  g = (g_a, g_b)
  sems = (sem_a, sem_b)

  pltpu.sync_copy(y_hbm.at[pl.ds(base, _BPW), :], yi_v)
  pltpu.sync_copy(h_hbm.at[pl.ds(base, _BPW), :], h_v)

  def build_idx(s, dst):
    ss = jnp.zeros((16,), jnp.int32) + s
    for j in range(_BPW // 16):
      dst[pl.ds(j * 16, 16)] = plsc.load_gather(yi_v, [j * 16 + iota, ss])

  build_idx(0, yidx_a)
  pltpu.async_copy(wpt_hbm.at[yidx_a], g_a, sem_a)

  for s in range(_S):
    cur = s % 2
    nxt = (s + 1) % 2
    if s + 1 < _S:
      build_idx(s + 1, yidx[nxt])
      pltpu.async_copy(wpt_hbm.at[yidx[nxt]], g[nxt], sems[nxt])
    pltpu.make_async_copy(wpt_hbm.at[yidx[cur]], g[cur], sems[cur]).wait()

    gc = g[cur]
    ss = jnp.zeros((16,), jnp.int32) + s

    def dot_row(i, _):
      acc = h_v[i, pl.ds(0, 16)] * gc[i, pl.ds(0, 16)]
      for j in range(1, _EMB // 16):
        sl = pl.ds(j * 16, 16)
        acc = acc + h_v[i, sl] * gc[i, sl]
      csum = plsc.cumsum(acc)
      plsc.store_scatter(out_v, [jnp.zeros((16,), jnp.int32) + i, ss],
                         csum, mask=lane15)
      return 0
    lax.fori_loop(0, _BPW, dot_row, 0)

  pltpu.sync_copy(out_v, out_hbm.at[pl.ds(base, _BPW), :])


@jax.jit
def _sc2_call(h, wpt, y):
  mesh = plsc.VectorSubcoreMesh(
      core_axis_name="c", subcore_axis_name="s",
      num_cores=_NC, num_subcores=_NS)
  return pl.kernel(
      _sc2_body,
      out_type=jax.ShapeDtypeStruct((_B, _S), jnp.float32),
      mesh=mesh,
      compiler_params=_SC_PARAMS,
      scratch_types=[
          pltpu.VMEM((_BPW, _S), jnp.int32),
          pltpu.VMEM((_BPW,), jnp.int32),
          pltpu.VMEM((_BPW,), jnp.int32),
          pltpu.VMEM((_BPW, _LANES), jnp.float32),
          pltpu.VMEM((_BPW, _LANES), jnp.float32),
          pltpu.VMEM((_BPW, _LANES), jnp.float32),
          pltpu.VMEM((_BPW, _S), jnp.float32),
          pltpu.SemaphoreType.DMA,
          pltpu.SemaphoreType.DMA,
      ],
  )(h, wpt, y)


def kernel(D, W, WP, ctxs, docs, y):
  w_lin = _to_rows(W.T)
  d_lin = _to_rows(D.T)
  h = _sc1_call(d_lin, w_lin, docs.astype(jnp.int32), ctxs.astype(jnp.int32))
  wpt = _to_rows(WP)
  return _sc2_call(h, wpt, y.astype(jnp.int32))


# speedup vs baseline: 10.0305x; 1.0594x over previous
import jax
import jax.numpy as jnp
from jax import lax
from jax.experimental import pallas as pl
from jax.experimental.pallas import tpu as pltpu
from jax.experimental.pallas import tpu_sc as plsc

_B = 4096
_CTX = 20
_S = 21
_EMB = 64
_LANES = 128
_VOCAB = 1000000
_NC = 2
_NS = 16
_NW = _NC * _NS
_BPW = _B // _NW
_CB = 16384

_SC_PARAMS = pltpu.CompilerParams(needs_layout_passes=False,
                                  use_tc_tiling_on_sc=False)


_NB2 = 31
_SPLIT = _NB2 * _CB


def _tr_body(x_ref, out_ref):
  out_ref[:, 0:_EMB] = x_ref[...].T
  out_ref[:, _EMB:_LANES] = jnp.zeros((_CB, _LANES - _EMB), jnp.float32)


def _tr2_body(x1_ref, x2_ref, out_ref):
  out_ref[:, 0:_EMB] = x1_ref[...].T
  out_ref[:, _EMB:_LANES] = x2_ref[...].T


def _to_rows(xt):
  n = xt.shape[1]
  nb = (n + _CB - 1) // _CB
  return pl.pallas_call(
      _tr_body,
      out_shape=jax.ShapeDtypeStruct((nb * _CB, _LANES), jnp.float32),
      grid=(nb,),
      in_specs=[pl.BlockSpec((_EMB, _CB), lambda i: (0, i))],
      out_specs=pl.BlockSpec((_CB, _LANES), lambda i: (i, 0)),
  )(xt)


def _to_rows_packed(xt):
  return pl.pallas_call(
      _tr2_body,
      out_shape=jax.ShapeDtypeStruct((_SPLIT, _LANES), jnp.float32),
      grid=(_NB2,),
      in_specs=[pl.BlockSpec((_EMB, _CB), lambda i: (0, i)),
                pl.BlockSpec((_EMB, _CB), lambda i: (0, i + _NB2))],
      out_specs=pl.BlockSpec((_CB, _LANES), lambda i: (i, 0)),
  )(xt, xt)


def _sc1_body(d_hbm, w_hbm, docs_hbm, ctxs_hbm, h_hbm,
              idx_a, idx_b, ident_v, ctxi_v, rows_a, rows_b, acc_spm,
              sem_a, sem_b):
  sid = lax.axis_index("s")
  wid = sid * _NC + lax.axis_index("c")
  base = wid * _BPW
  iota = lax.iota(jnp.int32, 16)
  sbase = sid * _BPW

  for j in range(_BPW // 16):
    ident_v[pl.ds(j * 16, 16)] = iota + (sbase + j * 16)

  idx = (idx_a, idx_b)
  rows = (rows_a, rows_b)
  sems = (sem_a, sem_b)

  pltpu.sync_copy(docs_hbm.at[pl.ds(base, _BPW)], idx_a)
  pltpu.async_copy(d_hbm.at[idx_a], rows_a, sem_a)
  pltpu.sync_copy(ctxs_hbm.at[pl.ds(base, _BPW), :], ctxi_v)

  def build_idx(c, dst):
    cc = jnp.zeros((16,), jnp.int32) + c
    for j in range(_BPW // 16):
      dst[pl.ds(j * 16, 16)] = plsc.load_gather(ctxi_v, [j * 16 + iota, cc])

  build_idx(0, idx_b)
  pltpu.make_async_copy(d_hbm.at[idx_a], rows_a, sem_a).wait()
  pltpu.sync_copy(rows_a, acc_spm.at[pl.ds(sbase, _BPW)])
  pltpu.async_copy(w_hbm.at[idx_b], rows_b, sem_b)

  for c in range(_CTX):
    cur = (c + 1) % 2
    nxt = c % 2
    if c + 1 < _CTX:
      build_idx(c + 1, idx[nxt])
      pltpu.async_copy(w_hbm.at[idx[nxt]], rows[nxt], sems[nxt])
    pltpu.make_async_copy(w_hbm.at[idx[cur]], rows[cur], sems[cur]).wait()
    pltpu.sync_copy(rows[cur], acc_spm.at[ident_v], add=True)

  pltpu.sync_copy(acc_spm.at[pl.ds(sbase, _BPW)],
                  h_hbm.at[pl.ds(base, _BPW), :])


@jax.jit
def _sc1_call(d, w, docs, ctxs):
  mesh = plsc.VectorSubcoreMesh(
      core_axis_name="c", subcore_axis_name="s",
      num_cores=_NC, num_subcores=_NS)
  return pl.kernel(
      _sc1_body,
      out_type=jax.ShapeDtypeStruct((_B, _LANES), jnp.float32),
      mesh=mesh,
      compiler_params=_SC_PARAMS,
      scratch_types=[
          pltpu.VMEM((_BPW,), jnp.int32),
          pltpu.VMEM((_BPW,), jnp.int32),
          pltpu.VMEM((_BPW,), jnp.int32),
          pltpu.VMEM((_BPW, _CTX), jnp.int32),
          pltpu.VMEM((_BPW, _LANES), jnp.float32),
          pltpu.VMEM((_BPW, _LANES), jnp.float32),
          pltpu.VMEM_SHARED((_NS * _BPW, _LANES), jnp.float32),
          pltpu.SemaphoreType.DMA,
          pltpu.SemaphoreType.DMA,
      ],
  )(d, w, docs, ctxs)


def _sc2_body(h_hbm, wpt_hbm, y_hbm, out_hbm,
              yi_v, yidx_a, yidx_b, h_v, g_a, g_b, out_v, sem_a, sem_b):
  sid = lax.axis_index("s")
  wid = sid * _NC + lax.axis_index("c")
  base = wid * _BPW
  iota = lax.iota(jnp.int32, 16)
  lane15 = iota == 15

  yidx = (yidx_a, yidx_b)
  g = (g_a, g_b)
  sems = (sem_a, sem_b)

  pltpu.sync_copy(y_hbm.at[pl.ds(base, _BPW), :], yi_v)
  pltpu.sync_copy(h_hbm.at[pl.ds(base, _BPW), :], h_v)

  def build_idx(s, dst):
    ss = jnp.zeros((16,), jnp.int32) + s
    for j in range(_BPW // 16):
      yv = plsc.load_gather(yi_v, [j * 16 + iota, ss])
      dst[pl.ds(j * 16, 16)] = jnp.where(yv >= _SPLIT, yv - _SPLIT, yv)

  build_idx(0, yidx_a)
  pltpu.async_copy(wpt_hbm.at[yidx_a], g_a, sem_a)

  for s in range(_S):
    cur = s % 2
    nxt = (s + 1) % 2
    if s + 1 < _S:
      build_idx(s + 1, yidx[nxt])
      pltpu.async_copy(wpt_hbm.at[yidx[nxt]], g[nxt], sems[nxt])
    pltpu.make_async_copy(wpt_hbm.at[yidx[cur]], g[cur], sems[cur]).wait()

    gc = g[cur]
    ss = jnp.zeros((16,), jnp.int32) + s

    def dot_row(i, _):
      ii = jnp.zeros((16,), jnp.int32) + i
      acc_lo = h_v[i, pl.ds(0, 16)] * gc[i, pl.ds(0, 16)]
      acc_hi = h_v[i, pl.ds(0, 16)] * gc[i, pl.ds(_EMB, 16)]
      for j in range(1, _EMB // 16):
        sl = pl.ds(j * 16, 16)
        acc_lo = acc_lo + h_v[i, sl] * gc[i, pl.ds(j * 16, 16)]
        acc_hi = acc_hi + h_v[i, sl] * gc[i, pl.ds(_EMB + j * 16, 16)]
      yv = plsc.load_gather(yi_v, [ii, ss])
      acc = jnp.where(yv >= _SPLIT, acc_hi, acc_lo)
      csum = plsc.cumsum(acc)
      plsc.store_scatter(out_v, [ii, ss], csum, mask=lane15)
      return 0
    lax.fori_loop(0, _BPW, dot_row, 0)

  pltpu.sync_copy(out_v, out_hbm.at[pl.ds(base, _BPW), :])


@jax.jit
def _sc2_call(h, wpt, y):
  mesh = plsc.VectorSubcoreMesh(
      core_axis_name="c", subcore_axis_name="s",
      num_cores=_NC, num_subcores=_NS)
  return pl.kernel(
      _sc2_body,
      out_type=jax.ShapeDtypeStruct((_B, _S), jnp.float32),
      mesh=mesh,
      compiler_params=_SC_PARAMS,
      scratch_types=[
          pltpu.VMEM((_BPW, _S), jnp.int32),
          pltpu.VMEM((_BPW,), jnp.int32),
          pltpu.VMEM((_BPW,), jnp.int32),
          pltpu.VMEM((_BPW, _LANES), jnp.float32),
          pltpu.VMEM((_BPW, _LANES), jnp.float32),
          pltpu.VMEM((_BPW, _LANES), jnp.float32),
          pltpu.VMEM((_BPW, _S), jnp.float32),
          pltpu.SemaphoreType.DMA,
          pltpu.SemaphoreType.DMA,
      ],
  )(h, wpt, y)


def kernel(D, W, WP, ctxs, docs, y):
  w_lin = _to_rows(W.T)
  d_lin = _to_rows(D.T)
  h = _sc1_call(d_lin, w_lin, docs.astype(jnp.int32), ctxs.astype(jnp.int32))
  wpt = _to_rows_packed(WP)
  return _sc2_call(h, wpt, y.astype(jnp.int32))
